# baseline (speedup 1.0000x reference)
import functools

import jax
import jax.numpy as jnp
import numpy as np
from jax import lax
from jax.experimental import pallas as pl
from jax.experimental.pallas import tpu as pltpu
from jax.experimental.pallas import tpu_sc as plsc

_N = 10000
_E = 320000

_NC = 2
_NS = 16
_NW = _NC * _NS
_B = 125
_KA = _E // (_NS * _B)
_KB = _E // (_NW * _B)
_RPS = _N // _NS


def _edge_agg(table, ei3, zeros, feature_split):
  mesh = plsc.VectorSubcoreMesh(core_axis_name="c", subcore_axis_name="s")
  k = _KA if feature_split else _KB

  @functools.partial(
      pl.kernel,
      out_type=jax.ShapeDtypeStruct((_N, 32), jnp.float32),
      mesh=mesh,
      compiler_params=pltpu.CompilerParams(use_tc_tiling_on_sc=False),
      scratch_types=[
          pltpu.VMEM((k, _B), jnp.int32),
          pltpu.VMEM((k, _B), jnp.int32),
          pltpu.VMEM((4, _B, 16), jnp.float32),
          pltpu.VMEM_SHARED((_N, 16), jnp.float32),
          pltpu.SemaphoreType.DMA((4,)),
          pltpu.SemaphoreType.DMA((4,)),
      ],
  )
  def agg(table_hbm, ei_hbm, zeros_hbm, out_hbm,
          src_v, dst_v, rows_v, acc_sh, gsem, ssem):
    c = lax.axis_index("c")
    s = lax.axis_index("s")
    rows = pl.ds(s * _RPS, _RPS)
    tbl = table_hbm.at[c] if feature_split else table_hbm
    widx = s if feature_split else s * _NC + c

    pltpu.sync_copy(zeros_hbm, acc_sh.at[rows])
    pltpu.sync_copy(ei_hbm.at[0, pl.ds(widx * k, k)], src_v)
    pltpu.sync_copy(ei_hbm.at[1, pl.ds(widx * k, k)], dst_v)
    plsc.subcore_barrier()

    def wait_dma(sem_slot, b):
      pltpu.make_async_copy(zeros_hbm.at[pl.ds(0, _B)], rows_v.at[b],
                            sem_slot).wait()

    for b in range(3):
      pltpu.async_copy(tbl.at[src_v.at[b]], rows_v.at[b], gsem.at[b])

    @pl.loop(0, k, step=4)
    def _(j):
      for b in range(4):
        jj = j + b
        wait_dma(gsem.at[b], b)
        pltpu.async_copy(rows_v.at[b], acc_sh.at[dst_v.at[jj]], ssem.at[b],
                         add=True)
        bb = (b + 3) % 4
        ja = jj + 3

        @pl.when(ja < k)
        def _(bb=bb, ja=ja, jj=jj):
          @pl.when(jj >= 1)
          def _():
            wait_dma(ssem.at[bb], bb)
          pltpu.async_copy(tbl.at[src_v.at[ja]], rows_v.at[bb],
                           gsem.at[bb])

    for b in range(4):
      wait_dma(ssem.at[b], b)
    plsc.subcore_barrier()
    pltpu.sync_copy(acc_sh.at[rows], out_hbm.at[rows, pl.ds(16 * c, 16)])

  return agg(table, ei3, zeros)


def _dot(a, b):
  return jnp.dot(a, b, preferred_element_type=jnp.float32)


def _proj(x, w1_blk4, sel_l, sel_r):
  def body(x_ref, w_ref, sl_ref, sr_ref, o_ref, oh_ref):
    x4 = jnp.reshape(x_ref[...], (_N // 4, 4 * 128))
    z = _dot(x4, w_ref[...])
    o_ref[...] = z
    z8 = jnp.reshape(z, (_N // 8, 256))
    oh_ref[0] = _dot(z8, sl_ref[...])
    oh_ref[1] = _dot(z8, sr_ref[...])
  return pl.pallas_call(
      body,
      out_shape=(
          jax.ShapeDtypeStruct((_N // 4, 128), jnp.float32),
          jax.ShapeDtypeStruct((2, _N // 8, 128), jnp.float32),
      ),
  )(x, w1_blk4, sel_l, sel_r)


def _mid(z_p, p_p, b1_t4, w2_blk8, b2_t8):
  def body(z_ref, p_ref, b1_ref, w2_ref, b2_ref, o_ref):
    t = jnp.maximum(z_ref[...] + p_ref[...] + b1_ref[...], 0.0)
    t8 = jnp.reshape(t, (_N // 8, 256))
    o_ref[...] = jnp.maximum(_dot(t8, w2_ref[...]) + b2_ref[...], 0.0)
  return pl.pallas_call(
      body,
      out_shape=jax.ShapeDtypeStruct((_N // 8, 128), jnp.float32),
  )(z_p, p_p, b1_t4, w2_blk8, b2_t8)


def _heads(h_p, q_p, sel, w1m_blk8, b1m_t8, w2m_blk8, b2m_t8,
           w1s_blk8, b1s_t8, w2s_blk8, b2s_t8):
  def body(h_ref, q_ref, sel_ref, w1m_ref, b1m_ref, w2m_ref, b2m_ref,
           w1s_ref, b1s_ref, w2s_ref, b2s_ref, mu_ref, ls_ref):
    q8 = jnp.reshape(q_ref[...], (_N // 8, 256))
    u = h_ref[...] + _dot(q8, sel_ref[...])
    tm = jnp.maximum(_dot(u, w1m_ref[...]) + b1m_ref[...], 0.0)
    mu_ref[...] = _dot(tm, w2m_ref[...]) + b2m_ref[...]
    ts = jnp.maximum(_dot(u, w1s_ref[...]) + b1s_ref[...], 0.0)
    ls_ref[...] = _dot(ts, w2s_ref[...]) + b2s_ref[...]
  return pl.pallas_call(
      body,
      out_shape=(
          jax.ShapeDtypeStruct((_N // 8, 128), jnp.float32),
          jax.ShapeDtypeStruct((_N // 8, 128), jnp.float32),
      ),
  )(h_p, q_p, sel, w1m_blk8, b1m_t8, w2m_blk8, b2m_t8,
    w1s_blk8, b1s_t8, w2s_blk8, b2s_t8)


def _blk(w, g):
  return jnp.kron(jnp.eye(g, dtype=jnp.float32), w)


_SEL_L = np.zeros((256, 128), np.float32)
_SEL_R = np.zeros((256, 128), np.float32)
for _q in range(8):
  for _c in range(16):
    _SEL_L[32 * _q + _c, 16 * _q + _c] = 1.0
    _SEL_R[32 * _q + 16 + _c, 16 * _q + _c] = 1.0
_SEL = _SEL_L + _SEL_R


@jax.jit
def kernel(x, edge_index, W1a, b1a, W2a, b2a, W1m, b1m, W2m, b2m,
           W1s, b1s, W2s, b2s):
  ei3 = edge_index.reshape(2, _E // _B, _B)
  zeros = jnp.zeros((_RPS, 16), jnp.float32)

  z_p, zh = _proj(x, _blk(W1a, 4), jnp.asarray(_SEL_L), jnp.asarray(_SEL_R))
  p = _edge_agg(zh.reshape(2, _N, 16), ei3, zeros, True)
  h_p = _mid(z_p, p.reshape(_N // 4, 128), jnp.tile(b1a, 4)[None],
             _blk(W2a, 8), jnp.tile(b2a, 8)[None])

  q = _edge_agg(h_p.reshape(_N, 16), ei3, zeros, False)
  mu_p, ls_p = _heads(h_p, q.reshape(_N // 4, 128), jnp.asarray(_SEL),
                      _blk(W1m, 8), jnp.tile(b1m, 8)[None],
                      _blk(W2m, 8), jnp.tile(b2m, 8)[None],
                      _blk(W1s, 8), jnp.tile(b1s, 8)[None],
                      _blk(W2s, 8), jnp.tile(b2s, 8)[None])
  return (mu_p.reshape(_N, 16), ls_p.reshape(_N, 16))


# speedup vs baseline: 1.2464x; 1.2464x over previous
import functools

import jax
import jax.numpy as jnp
import numpy as np
from jax import lax
from jax.experimental import pallas as pl
from jax.experimental.pallas import tpu as pltpu
from jax.experimental.pallas import tpu_sc as plsc

_N = 10000
_E = 320000

_NC = 2
_NS = 16
_NW = _NC * _NS
_B = 125
_KA = _E // (_NS * _B)
_KB = _E // (_NW * _B)
_RPS = _N // _NS


def _edge_agg(table, ei3, zeros, feature_split):
  mesh = plsc.VectorSubcoreMesh(core_axis_name="c", subcore_axis_name="s")
  k = _KA if feature_split else _KB

  @functools.partial(
      pl.kernel,
      out_type=jax.ShapeDtypeStruct((_N, 32), jnp.float32),
      mesh=mesh,
      compiler_params=pltpu.CompilerParams(use_tc_tiling_on_sc=False),
      scratch_types=[
          pltpu.VMEM((k, _B), jnp.int32),
          pltpu.VMEM((k, _B), jnp.int32),
          pltpu.VMEM((4, _B, 16), jnp.float32),
          pltpu.VMEM_SHARED((_N, 16), jnp.float32),
          pltpu.VMEM_SHARED((_N, 16), jnp.float32),
          pltpu.SemaphoreType.DMA((4,)),
          pltpu.SemaphoreType.DMA((4,)),
      ],
  )
  def agg(table_hbm, ei_hbm, zeros_hbm, out_hbm,
          src_v, dst_v, rows_v, tbl_sh, acc_sh, gsem, ssem):
    c = lax.axis_index("c")
    s = lax.axis_index("s")
    rows = pl.ds(s * _RPS, _RPS)
    tbl = tbl_sh
    widx = s if feature_split else s * _NC + c

    pltpu.sync_copy(zeros_hbm, acc_sh.at[rows])
    if feature_split:
      pltpu.sync_copy(table_hbm.at[c, rows], tbl_sh.at[rows])
    else:
      pltpu.sync_copy(table_hbm.at[rows], tbl_sh.at[rows])
    pltpu.sync_copy(ei_hbm.at[0, pl.ds(widx * k, k)], src_v)
    pltpu.sync_copy(ei_hbm.at[1, pl.ds(widx * k, k)], dst_v)
    plsc.subcore_barrier()

    def wait_dma(sem_slot, b):
      pltpu.make_async_copy(zeros_hbm.at[pl.ds(0, _B)], rows_v.at[b],
                            sem_slot).wait()

    for b in range(3):
      pltpu.async_copy(tbl.at[src_v.at[b]], rows_v.at[b], gsem.at[b])

    @pl.loop(0, k, step=4)
    def _(j):
      for b in range(4):
        jj = j + b
        wait_dma(gsem.at[b], b)
        pltpu.async_copy(rows_v.at[b], acc_sh.at[dst_v.at[jj]], ssem.at[b],
                         add=True)
        bb = (b + 3) % 4
        ja = jj + 3

        @pl.when(ja < k)
        def _(bb=bb, ja=ja, jj=jj):
          @pl.when(jj >= 1)
          def _():
            wait_dma(ssem.at[bb], bb)
          pltpu.async_copy(tbl.at[src_v.at[ja]], rows_v.at[bb],
                           gsem.at[bb])

    for b in range(4):
      wait_dma(ssem.at[b], b)
    plsc.subcore_barrier()
    pltpu.sync_copy(acc_sh.at[rows], out_hbm.at[rows, pl.ds(16 * c, 16)])

  return agg(table, ei3, zeros)


def _dot(a, b):
  return jnp.dot(a, b, preferred_element_type=jnp.float32)


def _proj(x, w1_blk4, sel_l, sel_r):
  def body(x_ref, w_ref, sl_ref, sr_ref, o_ref, oh_ref):
    x4 = jnp.reshape(x_ref[...], (_N // 4, 4 * 128))
    z = _dot(x4, w_ref[...])
    o_ref[...] = z
    z8 = jnp.reshape(z, (_N // 8, 256))
    oh_ref[0] = _dot(z8, sl_ref[...])
    oh_ref[1] = _dot(z8, sr_ref[...])
  return pl.pallas_call(
      body,
      out_shape=(
          jax.ShapeDtypeStruct((_N // 4, 128), jnp.float32),
          jax.ShapeDtypeStruct((2, _N // 8, 128), jnp.float32),
      ),
  )(x, w1_blk4, sel_l, sel_r)


def _mid(z_p, p_p, b1_t4, w2_blk8, b2_t8):
  def body(z_ref, p_ref, b1_ref, w2_ref, b2_ref, o_ref):
    t = jnp.maximum(z_ref[...] + p_ref[...] + b1_ref[...], 0.0)
    t8 = jnp.reshape(t, (_N // 8, 256))
    o_ref[...] = jnp.maximum(_dot(t8, w2_ref[...]) + b2_ref[...], 0.0)
  return pl.pallas_call(
      body,
      out_shape=jax.ShapeDtypeStruct((_N // 8, 128), jnp.float32),
  )(z_p, p_p, b1_t4, w2_blk8, b2_t8)


def _heads(h_p, q_p, sel, w1m_blk8, b1m_t8, w2m_blk8, b2m_t8,
           w1s_blk8, b1s_t8, w2s_blk8, b2s_t8):
  def body(h_ref, q_ref, sel_ref, w1m_ref, b1m_ref, w2m_ref, b2m_ref,
           w1s_ref, b1s_ref, w2s_ref, b2s_ref, mu_ref, ls_ref):
    q8 = jnp.reshape(q_ref[...], (_N // 8, 256))
    u = h_ref[...] + _dot(q8, sel_ref[...])
    tm = jnp.maximum(_dot(u, w1m_ref[...]) + b1m_ref[...], 0.0)
    mu_ref[...] = _dot(tm, w2m_ref[...]) + b2m_ref[...]
    ts = jnp.maximum(_dot(u, w1s_ref[...]) + b1s_ref[...], 0.0)
    ls_ref[...] = _dot(ts, w2s_ref[...]) + b2s_ref[...]
  return pl.pallas_call(
      body,
      out_shape=(
          jax.ShapeDtypeStruct((_N // 8, 128), jnp.float32),
          jax.ShapeDtypeStruct((_N // 8, 128), jnp.float32),
      ),
  )(h_p, q_p, sel, w1m_blk8, b1m_t8, w2m_blk8, b2m_t8,
    w1s_blk8, b1s_t8, w2s_blk8, b2s_t8)


def _blk(w, g):
  return jnp.kron(jnp.eye(g, dtype=jnp.float32), w)


_SEL_L = np.zeros((256, 128), np.float32)
_SEL_R = np.zeros((256, 128), np.float32)
for _q in range(8):
  for _c in range(16):
    _SEL_L[32 * _q + _c, 16 * _q + _c] = 1.0
    _SEL_R[32 * _q + 16 + _c, 16 * _q + _c] = 1.0
_SEL = _SEL_L + _SEL_R


@jax.jit
def kernel(x, edge_index, W1a, b1a, W2a, b2a, W1m, b1m, W2m, b2m,
           W1s, b1s, W2s, b2s):
  ei3 = edge_index.reshape(2, _E // _B, _B)
  zeros = jnp.zeros((_RPS, 16), jnp.float32)

  z_p, zh = _proj(x, _blk(W1a, 4), jnp.asarray(_SEL_L), jnp.asarray(_SEL_R))
  p = _edge_agg(zh.reshape(2, _N, 16), ei3, zeros, True)
  h_p = _mid(z_p, p.reshape(_N // 4, 128), jnp.tile(b1a, 4)[None],
             _blk(W2a, 8), jnp.tile(b2a, 8)[None])

  q = _edge_agg(h_p.reshape(_N, 16), ei3, zeros, False)
  mu_p, ls_p = _heads(h_p, q.reshape(_N // 4, 128), jnp.asarray(_SEL),
                      _blk(W1m, 8), jnp.tile(b1m, 8)[None],
                      _blk(W2m, 8), jnp.tile(b2m, 8)[None],
                      _blk(W1s, 8), jnp.tile(b1s, 8)[None],
                      _blk(W2s, 8), jnp.tile(b2s, 8)[None])
  return (mu_p.reshape(_N, 16), ls_p.reshape(_N, 16))


# speedup vs baseline: 1.2751x; 1.0230x over previous
import functools

import jax
import jax.numpy as jnp
import numpy as np
from jax import lax
from jax.experimental import pallas as pl
from jax.experimental.pallas import tpu as pltpu
from jax.experimental.pallas import tpu_sc as plsc

_N = 10000
_E = 320000

_NC = 2
_NS = 16
_NW = _NC * _NS
_B = 125
_KA = _E // (_NS * _B)
_KB = _E // (_NW * _B)
_RPS = _N // _NS


def _edge_agg(table, ei3, zeros, feature_split):
  mesh = plsc.VectorSubcoreMesh(core_axis_name="c", subcore_axis_name="s")
  k = _KA if feature_split else _KB

  @functools.partial(
      pl.kernel,
      out_type=jax.ShapeDtypeStruct((_N, 32), jnp.float32),
      mesh=mesh,
      compiler_params=pltpu.CompilerParams(use_tc_tiling_on_sc=False),
      scratch_types=[
          pltpu.VMEM((k, _B), jnp.int32),
          pltpu.VMEM((k, _B), jnp.int32),
          pltpu.VMEM((4, _B, 16), jnp.float32),
          pltpu.VMEM_SHARED((_N, 16), jnp.float32),
          pltpu.VMEM_SHARED((_N, 16), jnp.float32),
          pltpu.SemaphoreType.DMA((4,)),
          pltpu.SemaphoreType.DMA((4,)),
      ],
  )
  def agg(table_hbm, ei_hbm, zeros_hbm, out_hbm,
          src_v, dst_v, rows_v, tbl_sh, acc_sh, gsem, ssem):
    c = lax.axis_index("c")
    s = lax.axis_index("s")
    rows = pl.ds(s * _RPS, _RPS)
    widx = s if feature_split else s * _NC + c

    pltpu.sync_copy(zeros_hbm, acc_sh.at[rows])
    if feature_split:
      pltpu.sync_copy(table_hbm.at[rows, pl.ds(16 * c, 16)], tbl_sh.at[rows])
    else:
      pltpu.sync_copy(table_hbm.at[rows], tbl_sh.at[rows])
    pltpu.sync_copy(ei_hbm.at[0, pl.ds(widx * k, k)], src_v)
    pltpu.sync_copy(ei_hbm.at[1, pl.ds(widx * k, k)], dst_v)
    plsc.subcore_barrier()

    def wait_dma(sem_slot, b):
      pltpu.make_async_copy(zeros_hbm.at[pl.ds(0, _B)], rows_v.at[b],
                            sem_slot).wait()

    for b in range(3):
      pltpu.async_copy(tbl_sh.at[src_v.at[b]], rows_v.at[b], gsem.at[b])

    @pl.loop(0, k, step=4)
    def _(j):
      for b in range(4):
        jj = j + b
        wait_dma(gsem.at[b], b)
        pltpu.async_copy(rows_v.at[b], acc_sh.at[dst_v.at[jj]], ssem.at[b],
                         add=True)
        bb = (b + 3) % 4
        ja = jj + 3

        @pl.when(ja < k)
        def _(bb=bb, ja=ja, jj=jj):
          @pl.when(jj >= 1)
          def _():
            wait_dma(ssem.at[bb], bb)
          pltpu.async_copy(tbl_sh.at[src_v.at[ja]], rows_v.at[bb],
                           gsem.at[bb])

    for b in range(4):
      wait_dma(ssem.at[b], b)
    plsc.subcore_barrier()
    pltpu.sync_copy(acc_sh.at[rows], out_hbm.at[rows, pl.ds(16 * c, 16)])

  return agg(table, ei3, zeros)


def _dot(a, b):
  return jnp.dot(a, b, preferred_element_type=jnp.float32)


def _proj(x, w1_blk4):
  def body(x_ref, w_ref, o_ref):
    x4 = jnp.reshape(x_ref[...], (_N // 4, 4 * 128))
    o_ref[...] = _dot(x4, w_ref[...])
  return pl.pallas_call(
      body,
      out_shape=jax.ShapeDtypeStruct((_N // 4, 128), jnp.float32),
  )(x, w1_blk4)


def _mid(z_p, p_p, b1_t4, w2_blk8, b2_t8):
  def body(z_ref, p_ref, b1_ref, w2_ref, b2_ref, o_ref):
    t = jnp.maximum(z_ref[...] + p_ref[...] + b1_ref[...], 0.0)
    t8 = jnp.reshape(t, (_N // 8, 256))
    o_ref[...] = jnp.maximum(_dot(t8, w2_ref[...]) + b2_ref[...], 0.0)
  return pl.pallas_call(
      body,
      out_shape=jax.ShapeDtypeStruct((_N // 8, 128), jnp.float32),
  )(z_p, p_p, b1_t4, w2_blk8, b2_t8)


def _heads(h_p, q_p, sel, w1m_blk8, b1m_t8, w2m_blk8, b2m_t8,
           w1s_blk8, b1s_t8, w2s_blk8, b2s_t8):
  def body(h_ref, q_ref, sel_ref, w1m_ref, b1m_ref, w2m_ref, b2m_ref,
           w1s_ref, b1s_ref, w2s_ref, b2s_ref, mu_ref, ls_ref):
    q8 = jnp.reshape(q_ref[...], (_N // 8, 256))
    u = h_ref[...] + _dot(q8, sel_ref[...])
    tm = jnp.maximum(_dot(u, w1m_ref[...]) + b1m_ref[...], 0.0)
    mu_ref[...] = _dot(tm, w2m_ref[...]) + b2m_ref[...]
    ts = jnp.maximum(_dot(u, w1s_ref[...]) + b1s_ref[...], 0.0)
    ls_ref[...] = _dot(ts, w2s_ref[...]) + b2s_ref[...]
  return pl.pallas_call(
      body,
      out_shape=(
          jax.ShapeDtypeStruct((_N // 8, 128), jnp.float32),
          jax.ShapeDtypeStruct((_N // 8, 128), jnp.float32),
      ),
  )(h_p, q_p, sel, w1m_blk8, b1m_t8, w2m_blk8, b2m_t8,
    w1s_blk8, b1s_t8, w2s_blk8, b2s_t8)


def _blk(w, g):
  return jnp.kron(jnp.eye(g, dtype=jnp.float32), w)


_SEL_L = np.zeros((256, 128), np.float32)
_SEL_R = np.zeros((256, 128), np.float32)
for _q in range(8):
  for _c in range(16):
    _SEL_L[32 * _q + _c, 16 * _q + _c] = 1.0
    _SEL_R[32 * _q + 16 + _c, 16 * _q + _c] = 1.0
_SEL = _SEL_L + _SEL_R


@jax.jit
def kernel(x, edge_index, W1a, b1a, W2a, b2a, W1m, b1m, W2m, b2m,
           W1s, b1s, W2s, b2s):
  ei3 = edge_index.reshape(2, _E // _B, _B)
  zeros = jnp.zeros((_RPS, 16), jnp.float32)

  z_p = _proj(x, _blk(W1a, 4))
  p = _edge_agg(z_p.reshape(_N, 32), ei3, zeros, True)
  h_p = _mid(z_p, p.reshape(_N // 4, 128), jnp.tile(b1a, 4)[None],
             _blk(W2a, 8), jnp.tile(b2a, 8)[None])

  q = _edge_agg(h_p.reshape(_N, 16), ei3, zeros, False)
  mu_p, ls_p = _heads(h_p, q.reshape(_N // 4, 128), jnp.asarray(_SEL),
                      _blk(W1m, 8), jnp.tile(b1m, 8)[None],
                      _blk(W2m, 8), jnp.tile(b2m, 8)[None],
                      _blk(W1s, 8), jnp.tile(b1s, 8)[None],
                      _blk(W2s, 8), jnp.tile(b2s, 8)[None])
  return (mu_p.reshape(_N, 16), ls_p.reshape(_N, 16))
